# baseline (device time: 25884 ns/iter reference)
import jax
import jax.numpy as jnp
from jax import lax
from jax.experimental import pallas as pl
from jax.experimental.pallas import tpu as pltpu

E_LOCAL = 2
N_CHUNK = 2


def kernel(x, assign, W1, W2):
    tok, d = x.shape
    half = tok // 2
    chunk = half // N_CHUNK
    assign2 = assign.reshape(tok, 1)

    def body(x_ref, a_ref, w1_ref, w2_ref, out_ref,
             w1v, w2v, xstage, pstage, xsend, xrecv, asend, arecv,
             rsend, rrecv, osend, orecv, wsems, dsems, sems):
        my_x = lax.axis_index("x")
        my_y = lax.axis_index("y")
        ypeer = (my_x, 1 - my_y)
        xpeer = (1 - my_x, my_y)

        rows = pl.ds(my_x * half, half)
        xcopy = pltpu.make_async_copy(x_ref.at[rows, :], xstage, dsems.at[0])
        xcopy.start()
        wcopies = []
        for e in range(E_LOCAL):
            c1 = pltpu.make_async_copy(w1_ref.at[e], w1v.at[e], wsems.at[2 * e])
            c2 = pltpu.make_async_copy(w2_ref.at[e], w2v.at[e], wsems.at[2 * e + 1])
            c1.start()
            c2.start()
            wcopies.append((c1, c2))

        barrier = pltpu.get_barrier_semaphore()
        for nbr in (ypeer, xpeer):
            pl.semaphore_signal(barrier, inc=1, device_id=nbr,
                                device_id_type=pl.DeviceIdType.MESH)
        pl.semaphore_wait(barrier, 2)

        xcopy.wait()
        xsend[...] = xstage[...].astype(jnp.bfloat16)
        asend[...] = a_ref[rows, :]

        rdma_x = pltpu.make_async_remote_copy(
            src_ref=xsend, dst_ref=xrecv,
            send_sem=sems.at[0], recv_sem=sems.at[1],
            device_id=ypeer, device_id_type=pl.DeviceIdType.MESH)
        rdma_a = pltpu.make_async_remote_copy(
            src_ref=asend, dst_ref=arecv,
            send_sem=sems.at[2], recv_sem=sems.at[3],
            device_id=ypeer, device_id_type=pl.DeviceIdType.MESH)
        rdma_x.start()
        rdma_a.start()

        w1b = []
        w2b = []
        for e in range(E_LOCAL):
            c1, c2 = wcopies[e]
            c1.wait()
            c2.wait()
            w1b.append(w1v[e, :, :].astype(jnp.bfloat16))
            w2b.append(w2v[e, :, :].astype(jnp.bfloat16))

        def moe_local_experts(xb, a, m):
            acc = jnp.zeros((m, d), jnp.float32)
            for e in range(E_LOCAL):
                ge = my_y * E_LOCAL + e
                h = jnp.maximum(
                    jnp.dot(xb, w1b[e], preferred_element_type=jnp.float32),
                    0.0)
                y = jnp.dot(h.astype(jnp.bfloat16), w2b[e],
                            preferred_element_type=jnp.float32)
                acc = acc + jnp.where(a == ge, y, 0.0)
            return acc

        acc_mine = moe_local_experts(xsend[...], asend[...], half)

        rdma_x.wait()
        rdma_a.wait()

        rdmas_r = []
        for c in range(N_CHUNK):
            cs = pl.ds(c * chunk, chunk)
            rsend[cs, :] = moe_local_experts(
                xrecv[cs, :], arecv[cs, :], chunk).astype(jnp.bfloat16)
            rdma_r = pltpu.make_async_remote_copy(
                src_ref=rsend.at[cs, :], dst_ref=rrecv.at[cs, :],
                send_sem=sems.at[4 + c], recv_sem=sems.at[4 + N_CHUNK + c],
                device_id=ypeer, device_id_type=pl.DeviceIdType.MESH)
            rdma_r.start()
            rdmas_r.append(rdma_r)

        rdmas_o = []
        ocopies = []
        for c in range(N_CHUNK):
            cs = pl.ds(c * chunk, chunk)
            rdmas_r[c].wait_recv()
            myout = acc_mine[c * chunk:(c + 1) * chunk, :] + (
                rrecv[cs, :].astype(jnp.float32))
            osend[cs, :] = myout.astype(jnp.bfloat16)
            xstage[cs, :] = myout
            rdma_o = pltpu.make_async_remote_copy(
                src_ref=osend.at[cs, :], dst_ref=orecv.at[cs, :],
                send_sem=sems.at[4 + 2 * N_CHUNK + c],
                recv_sem=sems.at[4 + 3 * N_CHUNK + c],
                device_id=xpeer, device_id_type=pl.DeviceIdType.MESH)
            rdma_o.start()
            rdmas_o.append(rdma_o)
            oc = pltpu.make_async_copy(
                xstage.at[cs, :],
                out_ref.at[pl.ds(my_x * half + c * chunk, chunk), :],
                dsems.at[1 + c])
            oc.start()
            ocopies.append(oc)

        for c in range(N_CHUNK):
            cs = pl.ds(c * chunk, chunk)
            rdmas_o[c].wait_recv()
            pstage[cs, :] = orecv[cs, :].astype(jnp.float32)
            oc2 = pltpu.make_async_copy(
                pstage.at[cs, :],
                out_ref.at[pl.ds((1 - my_x) * half + c * chunk, chunk), :],
                dsems.at[1 + N_CHUNK + c])
            oc2.start()
            ocopies.append(oc2)

        for oc in ocopies:
            oc.wait()

        for c in range(N_CHUNK):
            rdmas_r[c].wait_send()
            rdmas_o[c].wait_send()

    return pl.pallas_call(
        body,
        out_shape=jax.ShapeDtypeStruct((tok, d), jnp.float32),
        in_specs=[
            pl.BlockSpec(memory_space=pl.ANY),
            pl.BlockSpec(memory_space=pltpu.VMEM),
            pl.BlockSpec(memory_space=pl.ANY),
            pl.BlockSpec(memory_space=pl.ANY),
        ],
        out_specs=pl.BlockSpec(memory_space=pl.ANY),
        scratch_shapes=[
            pltpu.VMEM(W1.shape, jnp.float32),
            pltpu.VMEM(W2.shape, jnp.float32),
            pltpu.VMEM((half, d), jnp.float32),
            pltpu.VMEM((half, d), jnp.float32),
            pltpu.VMEM((half, d), jnp.bfloat16),
            pltpu.VMEM((half, d), jnp.bfloat16),
            pltpu.VMEM((half, 1), jnp.int32),
            pltpu.VMEM((half, 1), jnp.int32),
            pltpu.VMEM((half, d), jnp.bfloat16),
            pltpu.VMEM((half, d), jnp.bfloat16),
            pltpu.VMEM((half, d), jnp.bfloat16),
            pltpu.VMEM((half, d), jnp.bfloat16),
            pltpu.SemaphoreType.DMA((2 * E_LOCAL,)),
            pltpu.SemaphoreType.DMA((1 + 2 * N_CHUNK,)),
            pltpu.SemaphoreType.DMA((4 + 4 * N_CHUNK,)),
        ],
        compiler_params=pltpu.CompilerParams(collective_id=0),
    )(x, assign2, W1, W2)


# device time: 25632 ns/iter; 1.0098x vs baseline; 1.0098x over previous
import jax
import jax.numpy as jnp
from jax import lax
from jax.experimental import pallas as pl
from jax.experimental.pallas import tpu as pltpu

E_LOCAL = 2
N_CHUNK = 2


def kernel(x, assign, W1, W2):
    tok, d = x.shape
    half = tok // 2
    chunk = half // N_CHUNK
    assign2 = assign.reshape(tok, 1)

    def body(x_ref, a_ref, w1_ref, w2_ref, out_ref,
             w1v, w2v, xstage, pstage, xsend, xrecv, asend, arecv,
             rsend, rrecv, osend, orecv, wsems, dsems, sems):
        my_x = lax.axis_index("x")
        my_y = lax.axis_index("y")
        ypeer = (my_x, 1 - my_y)
        xpeer = (1 - my_x, my_y)

        rows = pl.ds(my_x * half, half)
        xcopy = pltpu.make_async_copy(x_ref.at[rows, :], xstage, dsems.at[0])
        xcopy.start()
        wcopies = []
        for e in range(E_LOCAL):
            c1 = pltpu.make_async_copy(w1_ref.at[e], w1v.at[e], wsems.at[2 * e])
            c2 = pltpu.make_async_copy(w2_ref.at[e], w2v.at[e], wsems.at[2 * e + 1])
            c1.start()
            c2.start()
            wcopies.append((c1, c2))

        barrier = pltpu.get_barrier_semaphore()
        for nbr in (ypeer, xpeer):
            pl.semaphore_signal(barrier, inc=1, device_id=nbr,
                                device_id_type=pl.DeviceIdType.MESH)
        pl.semaphore_wait(barrier, 2)

        xcopy.wait()
        xsend[...] = xstage[...].astype(jnp.bfloat16)
        asend[...] = a_ref[rows, :]

        rdma_x = pltpu.make_async_remote_copy(
            src_ref=xsend, dst_ref=xrecv,
            send_sem=sems.at[0], recv_sem=sems.at[1],
            device_id=ypeer, device_id_type=pl.DeviceIdType.MESH)
        rdma_a = pltpu.make_async_remote_copy(
            src_ref=asend, dst_ref=arecv,
            send_sem=sems.at[2], recv_sem=sems.at[3],
            device_id=ypeer, device_id_type=pl.DeviceIdType.MESH)
        rdma_x.start()
        rdma_a.start()

        w1b = []
        w2b = []
        for e in range(E_LOCAL):
            c1, c2 = wcopies[e]
            c1.wait()
            c2.wait()
            w1b.append(w1v[e, :, :].astype(jnp.bfloat16))
            w2b.append(w2v[e, :, :].astype(jnp.bfloat16))

        def moe_local_experts(xb, a, m):
            acc = jnp.zeros((m, d), jnp.float32)
            for e in range(E_LOCAL):
                ge = my_y * E_LOCAL + e
                h = jnp.maximum(
                    jnp.dot(xb, w1b[e], preferred_element_type=jnp.float32),
                    0.0)
                y = jnp.dot(h.astype(jnp.bfloat16), w2b[e],
                            preferred_element_type=jnp.float32)
                acc = acc + jnp.where(a == ge, y, 0.0)
            return acc

        acc_mine = moe_local_experts(xsend[...], asend[...], half)

        rdma_x.wait()
        rdma_a.wait()

        rdmas_r = []
        for c in range(N_CHUNK):
            cs = pl.ds(c * chunk, chunk)
            rsend[cs, :] = moe_local_experts(
                xrecv[cs, :], arecv[cs, :], chunk).astype(jnp.bfloat16)
            rdma_r = pltpu.make_async_remote_copy(
                src_ref=rsend.at[cs, :], dst_ref=rrecv.at[cs, :],
                send_sem=sems.at[4 + c], recv_sem=sems.at[4 + N_CHUNK + c],
                device_id=ypeer, device_id_type=pl.DeviceIdType.MESH)
            rdma_r.start()
            rdmas_r.append(rdma_r)

        rdmas_o = []
        ocopies = []
        for c in range(N_CHUNK):
            cs = pl.ds(c * chunk, chunk)
            rdmas_r[c].wait_recv()
            myout = acc_mine[c * chunk:(c + 1) * chunk, :] + (
                rrecv[cs, :].astype(jnp.float32))
            osend[cs, :] = myout.astype(jnp.bfloat16)
            xstage[cs, :] = myout
            rdma_o = pltpu.make_async_remote_copy(
                src_ref=osend.at[cs, :], dst_ref=orecv.at[cs, :],
                send_sem=sems.at[4 + 2 * N_CHUNK + c],
                recv_sem=sems.at[4 + 3 * N_CHUNK + c],
                device_id=xpeer, device_id_type=pl.DeviceIdType.MESH)
            rdma_o.start()
            rdmas_o.append(rdma_o)
            oc = pltpu.make_async_copy(
                xstage.at[cs, :],
                out_ref.at[pl.ds(my_x * half + c * chunk, chunk), :],
                dsems.at[1 + c])
            oc.start()
            ocopies.append(oc)

        for c in range(N_CHUNK):
            cs = pl.ds(c * chunk, chunk)
            rdmas_o[c].wait_recv()
            pstage[cs, :] = orecv[cs, :].astype(jnp.float32)
            oc2 = pltpu.make_async_copy(
                pstage.at[cs, :],
                out_ref.at[pl.ds((1 - my_x) * half + c * chunk, chunk), :],
                dsems.at[1 + N_CHUNK + c])
            oc2.start()
            ocopies.append(oc2)

        for oc in ocopies:
            oc.wait()

        for c in range(N_CHUNK):
            rdmas_r[c].wait_send()
            rdmas_o[c].wait_send()

    return pl.pallas_call(
        body,
        out_shape=jax.ShapeDtypeStruct((tok, d), jnp.float32),
        in_specs=[
            pl.BlockSpec(memory_space=pl.ANY),
            pl.BlockSpec(memory_space=pltpu.VMEM),
            pl.BlockSpec(memory_space=pl.ANY),
            pl.BlockSpec(memory_space=pl.ANY),
        ],
        out_specs=pl.BlockSpec(memory_space=pl.ANY),
        scratch_shapes=[
            pltpu.VMEM(W1.shape, jnp.float32),
            pltpu.VMEM(W2.shape, jnp.float32),
            pltpu.VMEM((half, d), jnp.float32),
            pltpu.VMEM((half, d), jnp.float32),
            pltpu.VMEM((half, d), jnp.bfloat16),
            pltpu.VMEM((half, d), jnp.bfloat16),
            pltpu.VMEM((half, 1), jnp.int32),
            pltpu.VMEM((half, 1), jnp.int32),
            pltpu.VMEM((half, d), jnp.bfloat16),
            pltpu.VMEM((half, d), jnp.bfloat16),
            pltpu.VMEM((half, d), jnp.bfloat16),
            pltpu.VMEM((half, d), jnp.bfloat16),
            pltpu.SemaphoreType.DMA((2 * E_LOCAL,)),
            pltpu.SemaphoreType.DMA((1 + 2 * N_CHUNK,)),
            pltpu.SemaphoreType.DMA((4 + 4 * N_CHUNK,)),
        ],
        compiler_params=pltpu.CompilerParams(
            collective_id=0,
            vmem_limit_bytes=128 * 1024 * 1024,
        ),
    )(x, assign2, W1, W2)


# device time: 21260 ns/iter; 1.2175x vs baseline; 1.2056x over previous
import jax
import jax.numpy as jnp
from jax import lax
from jax.experimental import pallas as pl
from jax.experimental.pallas import tpu as pltpu

E_LOCAL = 2
N_CHUNK = 2


def kernel(x, assign, W1, W2):
    tok, d = x.shape
    half = tok // 2
    c0 = 5 * half // 8
    chunks = [(0, c0), (c0, half - c0)]
    assert len(chunks) == N_CHUNK

    def body(x_ref, a_ref, w1_ref, w2_ref, out_ref,
             w1v, w2v, xstage, pstage, xsend, xrecv, asend, arecv,
             rsend, ryprecv, rdgrecv, accsend, accrecv,
             wsems, dsems, sems):
        my_x = lax.axis_index("x")
        my_y = lax.axis_index("y")
        ypeer = (my_x, 1 - my_y)
        xpeer = (1 - my_x, my_y)
        diag = (1 - my_x, 1 - my_y)

        rows = pl.ds(my_x * half, half)
        xcopy = pltpu.make_async_copy(x_ref.at[rows, :], xstage, dsems.at[0])
        xcopy.start()
        wcopies = []
        for e in range(E_LOCAL):
            c1 = pltpu.make_async_copy(w1_ref.at[e], w1v.at[e], wsems.at[2 * e])
            c2 = pltpu.make_async_copy(w2_ref.at[e], w2v.at[e], wsems.at[2 * e + 1])
            c1.start()
            c2.start()
            wcopies.append((c1, c2))

        barrier = pltpu.get_barrier_semaphore()
        for nbr in (ypeer, xpeer, diag):
            pl.semaphore_signal(barrier, inc=1, device_id=nbr,
                                device_id_type=pl.DeviceIdType.MESH)
        pl.semaphore_wait(barrier, 3)

        xcopy.wait()
        xsend[...] = xstage[...].astype(jnp.bfloat16)
        asend[...] = a_ref[rows].reshape(half, 1)

        rdma_a = pltpu.make_async_remote_copy(
            src_ref=asend, dst_ref=arecv,
            send_sem=sems.at[0], recv_sem=sems.at[1],
            device_id=ypeer, device_id_type=pl.DeviceIdType.MESH)
        rdma_a.start()
        rdmas_x = []
        for c, (off, sz) in enumerate(chunks):
            cs = pl.ds(off, sz)
            rdma_x = pltpu.make_async_remote_copy(
                src_ref=xsend.at[cs, :], dst_ref=xrecv.at[cs, :],
                send_sem=sems.at[2 + c], recv_sem=sems.at[2 + N_CHUNK + c],
                device_id=ypeer, device_id_type=pl.DeviceIdType.MESH)
            rdma_x.start()
            rdmas_x.append(rdma_x)

        def expert_pass(xb, a, m, e, w1be, w2be):
            ge = my_y * E_LOCAL + e
            h = jnp.maximum(
                jnp.dot(xb, w1be, preferred_element_type=jnp.float32), 0.0)
            y = jnp.dot(h.astype(jnp.bfloat16), w2be,
                        preferred_element_type=jnp.float32)
            return jnp.where(a == ge, y, 0.0)

        w1b = [None] * E_LOCAL
        w2b = [None] * E_LOCAL
        xb_mine = xsend[...]
        a_mine = asend[...]
        acc_mine = jnp.zeros((half, d), jnp.float32)
        for e in range(E_LOCAL):
            c1, c2 = wcopies[e]
            c1.wait()
            w1b[e] = w1v[e, :, :].astype(jnp.bfloat16)
            c2.wait()
            w2b[e] = w2v[e, :, :].astype(jnp.bfloat16)
            acc_mine = acc_mine + expert_pass(
                xb_mine, a_mine, half, e, w1b[e], w2b[e])

        accsend[...] = acc_mine.astype(jnp.bfloat16)
        rdma_acc = pltpu.make_async_remote_copy(
            src_ref=accsend, dst_ref=accrecv,
            send_sem=sems.at[2 + 2 * N_CHUNK],
            recv_sem=sems.at[3 + 2 * N_CHUNK],
            device_id=xpeer, device_id_type=pl.DeviceIdType.MESH)
        rdma_acc.start()

        rdma_a.wait_recv()

        base = 4 + 2 * N_CHUNK
        rdmas_ryp = []
        rdmas_rdg = []
        for c, (off, sz) in enumerate(chunks):
            cs = pl.ds(off, sz)
            rdmas_x[c].wait_recv()
            racc = jnp.zeros((sz, d), jnp.float32)
            xb = xrecv[cs, :]
            a = arecv[cs, :]
            for e in range(E_LOCAL):
                racc = racc + expert_pass(xb, a, sz, e, w1b[e], w2b[e])
            rsend[cs, :] = racc.astype(jnp.bfloat16)
            r_yp = pltpu.make_async_remote_copy(
                src_ref=rsend.at[cs, :], dst_ref=ryprecv.at[cs, :],
                send_sem=sems.at[base + c],
                recv_sem=sems.at[base + N_CHUNK + c],
                device_id=ypeer, device_id_type=pl.DeviceIdType.MESH)
            r_yp.start()
            rdmas_ryp.append(r_yp)
            r_dg = pltpu.make_async_remote_copy(
                src_ref=rsend.at[cs, :], dst_ref=rdgrecv.at[cs, :],
                send_sem=sems.at[base + 2 * N_CHUNK + c],
                recv_sem=sems.at[base + 3 * N_CHUNK + c],
                device_id=diag, device_id_type=pl.DeviceIdType.MESH)
            r_dg.start()
            rdmas_rdg.append(r_dg)

        ocopies = []
        for c, (off, sz) in enumerate(chunks):
            cs = pl.ds(off, sz)
            rdmas_ryp[c].wait_recv()
            xstage[cs, :] = (
                acc_mine[off:off + sz, :]
                + ryprecv[cs, :].astype(jnp.float32))
            oc = pltpu.make_async_copy(
                xstage.at[cs, :],
                out_ref.at[pl.ds(my_x * half + off, sz), :],
                dsems.at[1 + c])
            oc.start()
            ocopies.append(oc)

        rdma_acc.wait_recv()
        for c, (off, sz) in enumerate(chunks):
            cs = pl.ds(off, sz)
            rdmas_rdg[c].wait_recv()
            pstage[cs, :] = (
                accrecv[cs, :].astype(jnp.float32)
                + rdgrecv[cs, :].astype(jnp.float32))
            oc = pltpu.make_async_copy(
                pstage.at[cs, :],
                out_ref.at[pl.ds((1 - my_x) * half + off, sz), :],
                dsems.at[1 + N_CHUNK + c])
            oc.start()
            ocopies.append(oc)
        for oc in ocopies:
            oc.wait()

        rdma_a.wait_send()
        rdma_acc.wait_send()
        for c in range(N_CHUNK):
            rdmas_x[c].wait_send()
            rdmas_ryp[c].wait_send()
            rdmas_rdg[c].wait_send()

    n_sems = 4 + 6 * N_CHUNK
    return pl.pallas_call(
        body,
        out_shape=jax.ShapeDtypeStruct((tok, d), jnp.float32),
        in_specs=[
            pl.BlockSpec(memory_space=pl.ANY),
            pl.BlockSpec(memory_space=pltpu.VMEM),
            pl.BlockSpec(memory_space=pl.ANY),
            pl.BlockSpec(memory_space=pl.ANY),
        ],
        out_specs=pl.BlockSpec(memory_space=pl.ANY),
        scratch_shapes=[
            pltpu.VMEM(W1.shape, jnp.float32),
            pltpu.VMEM(W2.shape, jnp.float32),
            pltpu.VMEM((half, d), jnp.float32),
            pltpu.VMEM((half, d), jnp.float32),
            pltpu.VMEM((half, d), jnp.bfloat16),
            pltpu.VMEM((half, d), jnp.bfloat16),
            pltpu.VMEM((half, 1), jnp.int32),
            pltpu.VMEM((half, 1), jnp.int32),
            pltpu.VMEM((half, d), jnp.bfloat16),
            pltpu.VMEM((half, d), jnp.bfloat16),
            pltpu.VMEM((half, d), jnp.bfloat16),
            pltpu.VMEM((half, d), jnp.bfloat16),
            pltpu.VMEM((half, d), jnp.bfloat16),
            pltpu.SemaphoreType.DMA((2 * E_LOCAL,)),
            pltpu.SemaphoreType.DMA((1 + 2 * N_CHUNK,)),
            pltpu.SemaphoreType.DMA((n_sems,)),
        ],
        compiler_params=pltpu.CompilerParams(
            collective_id=0,
            vmem_limit_bytes=128 * 1024 * 1024,
        ),
    )(x, assign, W1, W2)
